# baseline (speedup 1.0000x reference)
import dataclasses
import functools

import jax
import jax.numpy as jnp
from jax import lax
from jax.experimental import pallas as pl
from jax.experimental.pallas import tpu as pltpu
from jax.experimental.pallas import tpu_sc as plsc

N = 10000
E = 160000
D = 256
H = 256

NSUB = 16
EPS = E // NSUB
NHALF = N // 2
ACC_ROWS = NHALF + 8
CH = 64
LCAP = 5376
LSLACK = 64
PCH = 400
ROWS_PER_SUB = 312
WTAIL_BASE = NSUB * ROWS_PER_SUB
WTAIL = NHALF - WTAIL_BASE
NCHUNK = LCAP // CH
NPAIRS = NCHUNK // 2

_R = 1000
_GRID = N // _R


def _elu(v):
    return jnp.where(v > 0, v, jnp.exp(jnp.minimum(v, 0.0)) - 1.0)


def _bf16_bits(x):
    u = jax.lax.bitcast_convert_type(x, jnp.int32)
    r = (u + 0x7FFF + ((u >> 16) & 1)) >> 16
    return r & 0xFFFF


def _pack_bf16(s):
    half = s.shape[1] // 2
    return _bf16_bits(s[:, :half]) | (_bf16_bits(s[:, half:]) << 16)


def _mm1_body(x_ref, w_ref, o_ref):
    s = jnp.dot(x_ref[...], w_ref[...], preferred_element_type=jnp.float32)
    o_ref[...] = _pack_bf16(s)


def _mm_mid_body(a_ref, b_ref, w_ref, o_ref):
    f = _elu(a_ref[...] + b_ref[...])
    s = jnp.dot(f, w_ref[...], preferred_element_type=jnp.float32)
    o_ref[...] = _pack_bf16(s)


def _final_body(a_ref, b_ref, o_ref):
    z = a_ref[...] + b_ref[...]
    m = jnp.max(z, axis=1, keepdims=True)
    ez = jnp.exp(z - m)
    o_ref[...] = ez / jnp.sum(ez, axis=1, keepdims=True)


_table_t = jax.ShapeDtypeStruct((N, H // 2), jnp.int32)
_agg_t = jax.ShapeDtypeStruct((N, H), jnp.float32)

_mm1 = pl.pallas_call(
    _mm1_body,
    grid=(_GRID,),
    in_specs=[pl.BlockSpec((_R, D), lambda i: (i, 0)),
              pl.BlockSpec((D, H), lambda i: (0, 0))],
    out_specs=pl.BlockSpec((_R, H // 2), lambda i: (i, 0)),
    out_shape=_table_t,
)

_mm_mid = pl.pallas_call(
    _mm_mid_body,
    grid=(_GRID,),
    in_specs=[pl.BlockSpec((_R, H), lambda i: (i, 0)),
              pl.BlockSpec((1, H), lambda i: (0, 0)),
              pl.BlockSpec((H, H), lambda i: (0, 0))],
    out_specs=pl.BlockSpec((_R, H // 2), lambda i: (i, 0)),
    out_shape=_table_t,
)

_final = pl.pallas_call(
    _final_body,
    grid=(_GRID,),
    in_specs=[pl.BlockSpec((_R, H), lambda i: (i, 0)),
              pl.BlockSpec((1, H), lambda i: (0, 0))],
    out_specs=pl.BlockSpec((_R, H), lambda i: (i, 0)),
    out_shape=_agg_t,
)


def _part_body(src_hbm, dst_hbm, w_hbm, srcL, dstL, wL,
               src_l, dst_l, w_l,
               ps0, ps1, pd0, pd1, pw0, pw1,
               psem):
    c = lax.axis_index("c")
    s = lax.axis_index("s")
    pbuf = ((ps0, pd0, pw0), (ps1, pd1, pw1))

    ebase = s * EPS
    dlo = c * NHALF

    def start_pload(k, b):
        off = ebase + k * PCH
        pltpu.async_copy(src_hbm.at[pl.ds(off, PCH)], pbuf[b][0], psem)
        pltpu.async_copy(dst_hbm.at[pl.ds(off, PCH)], pbuf[b][1], psem)
        pltpu.async_copy(w_hbm.at[pl.ds(off, PCH)], pbuf[b][2], psem)

    def wait_pload(k, b):
        off = ebase + k * PCH
        pltpu.make_async_copy(src_hbm.at[pl.ds(off, PCH)], pbuf[b][0], psem).wait()
        pltpu.make_async_copy(dst_hbm.at[pl.ds(off, PCH)], pbuf[b][1], psem).wait()
        pltpu.make_async_copy(w_hbm.at[pl.ds(off, PCH)], pbuf[b][2], psem).wait()

    NP = EPS // PCH

    def compact(b, p0):
        def group(g, p):
            s16 = pbuf[b][0][pl.ds(g * 16, 16)]
            d16 = pbuf[b][1][pl.ds(g * 16, 16)]
            w16 = pbuf[b][2][pl.ds(g * 16, 16)]
            m = (d16 >= dlo) & (d16 < dlo + NHALF)
            plsc.store_compressed(src_l.at[pl.ds(p, 16)], s16, mask=m)
            plsc.store_compressed(dst_l.at[pl.ds(p, 16)], d16 - dlo, mask=m)
            plsc.store_compressed(w_l.at[pl.ds(p, 16)], w16, mask=m)
            cnt = plsc.all_reduce_population_count(m)
            return jnp.minimum(p + cnt[0], LCAP - 16)

        return pl.loop(0, PCH // 16, init_carry=p0)(group)

    start_pload(0, 0)
    p = jnp.int32(0)
    for k in range(NP):
        b = k % 2
        wait_pload(k, b)
        if k + 1 < NP:
            start_pload(k + 1, 1 - b)
        p = compact(b, p)

    @pl.loop(0, (LCAP - p + 15) // 16)
    def _(i):
        src_l[pl.ds(p + 16 * i, 16)] = jnp.zeros((16,), jnp.int32)
        dst_l[pl.ds(p + 16 * i, 16)] = jnp.full((16,), NHALF, jnp.int32)
        w_l[pl.ds(p + 16 * i, 16)] = jnp.zeros((16,), jnp.float32)

    lbase = (c * NSUB + s) * LCAP
    pltpu.sync_copy(src_l.at[pl.ds(0, LCAP)], srcL.at[pl.ds(lbase, LCAP)])
    pltpu.sync_copy(dst_l.at[pl.ds(0, LCAP)], dstL.at[pl.ds(lbase, LCAP)])
    pltpu.sync_copy(w_l.at[pl.ds(0, LCAP)], wL.at[pl.ds(lbase, LCAP)])


def _spmm_body(table, srcL, dstL, wL, out,
               acc, src_l,
               rows0, rows1, d0, d1, w0, w1, send_a, send_b,
               lsem, gs0, gs1):
    c = lax.axis_index("c")
    s = lax.axis_index("s")
    rows = (rows0, rows1)
    dbuf = (d0, d1)
    wbuf = (w0, w1)
    gsem = (gs0, gs1)

    lbase = (c * NSUB + s) * LCAP
    pltpu.async_copy(srcL.at[pl.ds(lbase, LCAP)], src_l, lsem)

    @pl.loop(0, CH)
    def _(r):
        for j in range(H // 32):
            send_a[r, pl.ds(16 * j, 16)] = jnp.zeros((16,), jnp.float32)

    rb = s * ROWS_PER_SUB
    for h in range(2):
        for i in range(ROWS_PER_SUB // CH):
            pltpu.sync_copy(send_a, acc.at[h, pl.ds(rb + i * CH, CH)])
        zrem = ROWS_PER_SUB - (ROWS_PER_SUB // CH) * CH
        if zrem:
            pltpu.sync_copy(send_a.at[pl.ds(0, zrem)],
                            acc.at[h, pl.ds(rb + ROWS_PER_SUB - zrem, zrem)])

        @pl.when(s == NSUB - 1)
        def _():
            pltpu.sync_copy(send_a.at[pl.ds(0, WTAIL + 8)],
                            acc.at[h, pl.ds(WTAIL_BASE, WTAIL + 8)])

    pltpu.make_async_copy(srcL.at[pl.ds(lbase, LCAP)], src_l, lsem).wait()
    plsc.subcore_barrier()

    def start_gather(k, b):
        pltpu.async_copy(table.at[src_l.at[pl.ds(k * CH, CH)]],
                         rows[b], gsem[b])
        pltpu.async_copy(dstL.at[pl.ds(lbase + k * CH, CH)],
                         dbuf[b], gsem[b])
        pltpu.async_copy(wL.at[pl.ds(lbase + k * CH, CH)],
                         wbuf[b], gsem[b])

    def wait_gather(k, b):
        pltpu.make_async_copy(table.at[src_l.at[pl.ds(k * CH, CH)]],
                              rows[b], gsem[b]).wait()
        pltpu.make_async_copy(dstL.at[pl.ds(lbase + k * CH, CH)],
                              dbuf[b], gsem[b]).wait()
        pltpu.make_async_copy(wL.at[pl.ds(lbase + k * CH, CH)],
                              wbuf[b], gsem[b]).wait()

    def process(k, b, last_lap):
        @pl.when(jnp.logical_not(last_lap))
        def _():
            start_gather(k + 1, 1 - b)

        wait_gather(k, b)

        @pl.loop(0, CH // 16)
        def _(g):
            w16 = wbuf[b][pl.ds(g * 16, 16)]
            for i in range(16):
                we = w16[i]
                e = g * 16 + i
                for j in range(H // 32):
                    v = rows[b][e, pl.ds(16 * j, 16)]
                    va = jax.lax.bitcast_convert_type(v << 16, jnp.float32)
                    vb = jax.lax.bitcast_convert_type(
                        v & jnp.int32(-65536), jnp.float32)
                    send_a[e, pl.ds(16 * j, 16)] = va * we
                    send_b[e, pl.ds(16 * j, 16)] = vb * we

        pltpu.sync_copy(send_a, acc.at[0].at[dbuf[b]], add=True)
        pltpu.sync_copy(send_b, acc.at[1].at[dbuf[b]], add=True)

    start_gather(0, 0)

    @pl.loop(0, NPAIRS)
    def _(t):
        k = t * 2
        process(k, 0, False)
        process(k + 1, 1, t == NPAIRS - 1)

    plsc.subcore_barrier()

    obase = c * NHALF
    for h in range(2):
        pltpu.sync_copy(
            acc.at[h, pl.ds(rb, ROWS_PER_SUB)],
            out.at[pl.ds(obase + rb, ROWS_PER_SUB), pl.ds(h * (H // 2), H // 2)])

    @pl.when(s == NSUB - 1)
    def _():
        for h in range(2):
            pltpu.sync_copy(
                acc.at[h, pl.ds(WTAIL_BASE, WTAIL)],
                out.at[pl.ds(obase + WTAIL_BASE, WTAIL),
                       pl.ds(h * (H // 2), H // 2)])


_part_params = pltpu.CompilerParams()
if "needs_layout_passes" in pltpu.CompilerParams.__dataclass_fields__:
    _part_params = dataclasses.replace(_part_params, needs_layout_passes=False)

_mesh = plsc.VectorSubcoreMesh(core_axis_name="c", subcore_axis_name="s",
                               num_cores=2, num_subcores=NSUB)

_list_t = jax.ShapeDtypeStruct((2 * NSUB * LCAP,), jnp.int32)
_listf_t = jax.ShapeDtypeStruct((2 * NSUB * LCAP,), jnp.float32)

_part = pl.kernel(
    _part_body,
    out_type=(_list_t, _list_t, _listf_t),
    compiler_params=_part_params,
    mesh=_mesh,
    scratch_types=[
        pltpu.VMEM((LCAP + LSLACK,), jnp.int32),
        pltpu.VMEM((LCAP + LSLACK,), jnp.int32),
        pltpu.VMEM((LCAP + LSLACK,), jnp.float32),
        pltpu.VMEM((PCH,), jnp.int32),
        pltpu.VMEM((PCH,), jnp.int32),
        pltpu.VMEM((PCH,), jnp.int32),
        pltpu.VMEM((PCH,), jnp.int32),
        pltpu.VMEM((PCH,), jnp.float32),
        pltpu.VMEM((PCH,), jnp.float32),
        pltpu.SemaphoreType.DMA,
    ],
)

_spmm = pl.kernel(
    _spmm_body,
    out_type=_agg_t,
    mesh=_mesh,
    scratch_types=[
        pltpu.VMEM_SHARED((2, ACC_ROWS, H // 2), jnp.float32),
        pltpu.VMEM((LCAP,), jnp.int32),
        pltpu.VMEM((CH, H // 2), jnp.int32),
        pltpu.VMEM((CH, H // 2), jnp.int32),
        pltpu.VMEM((CH,), jnp.int32),
        pltpu.VMEM((CH,), jnp.int32),
        pltpu.VMEM((CH,), jnp.float32),
        pltpu.VMEM((CH,), jnp.float32),
        pltpu.VMEM((CH, H // 2), jnp.float32),
        pltpu.VMEM((CH, H // 2), jnp.float32),
        pltpu.SemaphoreType.DMA,
        pltpu.SemaphoreType.DMA,
        pltpu.SemaphoreType.DMA,
    ],
)


def kernel(x, edge_index, edge_weight, W1, b1, W2, b2, W3, b3):
    dst = edge_index[0]
    src = edge_index[1]
    b1r = b1.reshape(1, H)
    b2r = b2.reshape(1, H)
    b3r = b3.reshape(1, H)

    srcL, dstL, wL = _part(src, dst, edge_weight)
    t = _mm1(x, W1)
    a = _spmm(t, srcL, dstL, wL)
    t = _mm_mid(a, b1r, W2)
    a = _spmm(t, srcL, dstL, wL)
    t = _mm_mid(a, b2r, W3)
    a = _spmm(t, srcL, dstL, wL)
    return _final(a, b3r)


# speedup vs baseline: 4.9749x; 4.9749x over previous
import jax
import jax.numpy as jnp
from jax import lax
from jax.experimental import pallas as pl
from jax.experimental.pallas import tpu as pltpu
from jax.experimental.pallas import tpu_sc as plsc

N = 10000
E = 160000
D = 256
H = 256
HH = 128

NSUB = 16
EPS = E // NSUB
CH = 80
NFULL = EPS // CH
ROWS_PER_SUB = 624
TAIL_BASE = NSUB * ROWS_PER_SUB
TAIL_ROWS = N - TAIL_BASE

_R = 1000
_GRID = N // _R


def _elu(v):
    return jnp.where(v > 0, v, jnp.exp(jnp.minimum(v, 0.0)) - 1.0)


def _mm1_body(x_ref, w_ref, o0_ref, o1_ref):
    s = jnp.dot(x_ref[...], w_ref[...], preferred_element_type=jnp.float32)
    o0_ref[...] = s[:, :HH]
    o1_ref[...] = s[:, HH:]


def _mm_mid_body(a0_ref, a1_ref, b_ref, w_ref, o0_ref, o1_ref):
    f0 = _elu(a0_ref[...] + b_ref[:, :HH])
    f1 = _elu(a1_ref[...] + b_ref[:, HH:])
    s = (jnp.dot(f0, w_ref[:HH, :], preferred_element_type=jnp.float32)
         + jnp.dot(f1, w_ref[HH:, :], preferred_element_type=jnp.float32))
    o0_ref[...] = s[:, :HH]
    o1_ref[...] = s[:, HH:]


def _final_body(a0_ref, a1_ref, b_ref, o_ref):
    z0 = a0_ref[...] + b_ref[:, :HH]
    z1 = a1_ref[...] + b_ref[:, HH:]
    z = jnp.concatenate([z0, z1], axis=1)
    m = jnp.max(z, axis=1, keepdims=True)
    ez = jnp.exp(z - m)
    o_ref[...] = ez / jnp.sum(ez, axis=1, keepdims=True)


_half = jax.ShapeDtypeStruct((N, HH), jnp.float32)

_mm1 = pl.pallas_call(
    _mm1_body,
    grid=(_GRID,),
    in_specs=[pl.BlockSpec((_R, D), lambda i: (i, 0)),
              pl.BlockSpec((D, H), lambda i: (0, 0))],
    out_specs=[pl.BlockSpec((_R, HH), lambda i: (i, 0)),
               pl.BlockSpec((_R, HH), lambda i: (i, 0))],
    out_shape=[_half, _half],
)

_mm_mid = pl.pallas_call(
    _mm_mid_body,
    grid=(_GRID,),
    in_specs=[pl.BlockSpec((_R, HH), lambda i: (i, 0)),
              pl.BlockSpec((_R, HH), lambda i: (i, 0)),
              pl.BlockSpec((1, H), lambda i: (0, 0)),
              pl.BlockSpec((H, H), lambda i: (0, 0))],
    out_specs=[pl.BlockSpec((_R, HH), lambda i: (i, 0)),
               pl.BlockSpec((_R, HH), lambda i: (i, 0))],
    out_shape=[_half, _half],
)

_final = pl.pallas_call(
    _final_body,
    grid=(_GRID,),
    in_specs=[pl.BlockSpec((_R, HH), lambda i: (i, 0)),
              pl.BlockSpec((_R, HH), lambda i: (i, 0)),
              pl.BlockSpec((1, H), lambda i: (0, 0))],
    out_specs=pl.BlockSpec((_R, H), lambda i: (i, 0)),
    out_shape=jax.ShapeDtypeStruct((N, H), jnp.float32),
)


def _spmm_body(t0, t1, src_hbm, dst_hbm, w_hbm, out0, out1,
               acc, src_v,
               rows0, rows1, rows2, d0, d1, d2, w0, w1, w2,
               gs0, gs1, gs2, ss0, ss1, ss2):
    c = lax.axis_index("c")
    s = lax.axis_index("s")
    rows = (rows0, rows1, rows2)
    dbuf = (d0, d1, d2)
    wbuf = (w0, w1, w2)
    gsem = (gs0, gs1, gs2)
    ssem = (ss0, ss1, ss2)

    @pl.loop(0, CH)
    def _(r):
        for j in range(HH // 16):
            rows0[r, pl.ds(16 * j, 16)] = jnp.zeros((16,), jnp.float32)

    rb = s * ROWS_PER_SUB
    for i in range(ROWS_PER_SUB // CH):
        pltpu.sync_copy(rows0, acc.at[pl.ds(rb + i * CH, CH)])
    rem = ROWS_PER_SUB - (ROWS_PER_SUB // CH) * CH
    if rem:
        pltpu.sync_copy(rows0.at[pl.ds(0, rem)],
                        acc.at[pl.ds(rb + ROWS_PER_SUB - rem, rem)])

    @pl.when(s == NSUB - 1)
    def _():
        pltpu.sync_copy(rows0.at[pl.ds(0, TAIL_ROWS)],
                        acc.at[pl.ds(TAIL_BASE, TAIL_ROWS)])

    ebase = s * EPS
    pltpu.sync_copy(src_hbm.at[pl.ds(ebase, EPS)], src_v)
    plsc.subcore_barrier()

    def start_gather(k, b):
        idx = src_v.at[pl.ds(k * CH, CH)]

        @pl.when(c == 0)
        def _():
            pltpu.async_copy(t0.at[idx], rows[b], gsem[b])

        @pl.when(c == 1)
        def _():
            pltpu.async_copy(t1.at[idx], rows[b], gsem[b])

        pltpu.async_copy(dst_hbm.at[pl.ds(ebase + k * CH, CH)],
                         dbuf[b], gsem[b])
        pltpu.async_copy(w_hbm.at[pl.ds(ebase + k * CH, CH)],
                         wbuf[b], gsem[b])

    def scale(buf, wsrc, nedge):
        @pl.loop(0, nedge // 16)
        def _(g):
            w16 = wsrc[pl.ds(g * 16, 16)]
            for i in range(16):
                we = w16[i]
                e = g * 16 + i
                for j in range(HH // 16):
                    sl = (pl.ds(e, 1), pl.ds(16 * j, 16))
                    buf[sl] = buf[sl] * we

    def wait_scatter(b):
        pltpu.make_async_copy(rows[b], acc.at[dbuf[b]], ssem[b]).wait()

    def process(k, b):
        nk = k + 1
        nb = (b + 1) % 3

        @pl.when(nk < NFULL)
        def _():
            @pl.when(k >= 2)
            def _():
                wait_scatter(nb)

            start_gather(nk, nb)

        pltpu.make_async_copy(t0.at[src_v.at[pl.ds(k * CH, CH)]],
                              rows[b], gsem[b]).wait()
        pltpu.make_async_copy(dst_hbm.at[pl.ds(ebase + k * CH, CH)],
                              dbuf[b], gsem[b]).wait()
        pltpu.make_async_copy(w_hbm.at[pl.ds(ebase + k * CH, CH)],
                              wbuf[b], gsem[b]).wait()
        scale(rows[b], wbuf[b], CH)
        pltpu.async_copy(rows[b], acc.at[dbuf[b]], ssem[b], add=True)

    start_gather(0, 0)

    @pl.loop(0, NFULL // 3)
    def _(t):
        k = t * 3
        process(k, 0)
        process(k + 1, 1)
        process(k + 2, 2)

    process(NFULL - 2, (NFULL - 2) % 3)
    process(NFULL - 1, (NFULL - 1) % 3)
    for b in range(3):
        wait_scatter(b)

    plsc.subcore_barrier()

    def writeout(out):
        WR = ROWS_PER_SUB // 2
        for i in range(2):
            ofs = rb + i * WR
            pltpu.sync_copy(acc.at[pl.ds(ofs, WR)],
                            out.at[pl.ds(ofs, WR)])

        @pl.when(s == NSUB - 1)
        def _():
            pltpu.sync_copy(acc.at[pl.ds(TAIL_BASE, TAIL_ROWS)],
                            out.at[pl.ds(TAIL_BASE, TAIL_ROWS)])

    @pl.when(c == 0)
    def _():
        writeout(out0)

    @pl.when(c == 1)
    def _():
        writeout(out1)


_spmm = pl.kernel(
    _spmm_body,
    out_type=(_half, _half),
    mesh=plsc.VectorSubcoreMesh(core_axis_name="c", subcore_axis_name="s",
                                num_cores=2, num_subcores=NSUB),
    scratch_types=[
        pltpu.VMEM_SHARED((N, HH), jnp.float32),
        pltpu.VMEM((EPS,), jnp.int32),
        pltpu.VMEM((CH, HH), jnp.float32),
        pltpu.VMEM((CH, HH), jnp.float32),
        pltpu.VMEM((CH, HH), jnp.float32),
        pltpu.VMEM((CH,), jnp.int32),
        pltpu.VMEM((CH,), jnp.int32),
        pltpu.VMEM((CH,), jnp.int32),
        pltpu.VMEM((CH,), jnp.float32),
        pltpu.VMEM((CH,), jnp.float32),
        pltpu.VMEM((CH,), jnp.float32),
        pltpu.SemaphoreType.DMA,
        pltpu.SemaphoreType.DMA,
        pltpu.SemaphoreType.DMA,
        pltpu.SemaphoreType.DMA,
        pltpu.SemaphoreType.DMA,
        pltpu.SemaphoreType.DMA,
    ],
)


def kernel(x, edge_index, edge_weight, W1, b1, W2, b2, W3, b3):
    dst = edge_index[0]
    src = edge_index[1]
    b1r = b1.reshape(1, H)
    b2r = b2.reshape(1, H)
    b3r = b3.reshape(1, H)

    s0, s1 = _mm1(x, W1)
    a0, a1 = _spmm(s0, s1, src, dst, edge_weight)
    s0, s1 = _mm_mid(a0, a1, b1r, W2)
    a0, a1 = _spmm(s0, s1, src, dst, edge_weight)
    s0, s1 = _mm_mid(a0, a1, b2r, W3)
    a0, a1 = _spmm(s0, s1, src, dst, edge_weight)
    return _final(a0, a1, b3r)
